# Initial kernel scaffold; baseline (speedup 1.0000x reference)
import jax
import jax.numpy as jnp
from jax.experimental import pallas as pl


def kernel(x, W, b, gumbel):
    raise NotImplementedError("write your pallas kernel here")


# speedup vs baseline: 1.6381x; 1.6381x over previous
import functools

import jax
import jax.numpy as jnp
from jax import lax
from jax.experimental import pallas as pl
from jax.experimental.pallas import tpu as pltpu

_TAU = 1.0
_K = 8


def _router_body(x_ref, w_ref, b_ref, g_ref, probs_ref, idx_ref, loss_ref,
                 cnt_ref, *, n_tokens, n_experts):
    pid = pl.program_id(0)

    @pl.when(pid == 0)
    def _init():
        cnt_ref[...] = jnp.zeros_like(cnt_ref)

    z = jnp.dot(x_ref[...], w_ref[...], preferred_element_type=jnp.float32)
    z = (z + b_ref[...] + g_ref[...]) * (1.0 / _TAU)

    iota = lax.broadcasted_iota(jnp.int32, z.shape, 1)
    vals = z
    sel = jnp.zeros(z.shape, jnp.float32)
    top_v = []
    top_i = []
    for _ in range(_K):
        m = jnp.max(vals, axis=1, keepdims=True)
        idx = jnp.min(jnp.where(vals == m, iota, n_experts), axis=1,
                      keepdims=True)
        top_v.append(m)
        top_i.append(idx)
        hit = iota == idx
        vals = jnp.where(hit, -jnp.inf, vals)
        sel = sel + hit.astype(jnp.float32)

    cnt_ref[...] += jnp.sum(sel, axis=0, keepdims=True)

    vtop = jnp.concatenate(top_v, axis=1)
    itop = jnp.concatenate(top_i, axis=1)
    e = jnp.exp(vtop - top_v[0])
    probs_ref[...] = e / jnp.sum(e, axis=1, keepdims=True)
    idx_ref[...] = itop

    @pl.when(pid == pl.num_programs(0) - 1)
    def _loss():
        counts = cnt_ref[...]
        total = jnp.float32(n_tokens * _K)
        expert_probs = counts / total
        t = jnp.float32(1.0 / n_experts)
        kl = jnp.sum(t * (jnp.log(t) - jnp.log(expert_probs)),
                     keepdims=True)
        loss_ref[...] = kl.reshape(1, 1) / n_experts


def kernel(x, W, b, gumbel):
    B, S, D = x.shape
    E = W.shape[1]
    T = B * S
    x2 = x.reshape(T, D)
    g2 = gumbel.reshape(T, E)
    b2 = b.reshape(1, E)

    bt = 512
    while T % bt:
        bt //= 2
    grid = T // bt

    body = functools.partial(_router_body, n_tokens=T, n_experts=E)
    probs, idx, loss = pl.pallas_call(
        body,
        grid=(grid,),
        in_specs=[
            pl.BlockSpec((bt, D), lambda i: (i, 0)),
            pl.BlockSpec((D, E), lambda i: (0, 0)),
            pl.BlockSpec((1, E), lambda i: (0, 0)),
            pl.BlockSpec((bt, E), lambda i: (i, 0)),
        ],
        out_specs=[
            pl.BlockSpec((bt, _K), lambda i: (i, 0)),
            pl.BlockSpec((bt, _K), lambda i: (i, 0)),
            pl.BlockSpec((1, 1), lambda i: (0, 0)),
        ],
        out_shape=[
            jax.ShapeDtypeStruct((T, _K), jnp.float32),
            jax.ShapeDtypeStruct((T, _K), jnp.int32),
            jax.ShapeDtypeStruct((1, 1), jnp.float32),
        ],
        scratch_shapes=[pltpu.VMEM((1, E), jnp.float32)],
        compiler_params=pltpu.CompilerParams(
            dimension_semantics=("arbitrary",),
        ),
    )(x2, W, b2, g2)

    return (probs.reshape(B, S, _K), idx.reshape(B, S, _K),
            loss.reshape(()))


# speedup vs baseline: 1.8050x; 1.1019x over previous
import functools

import jax
import jax.numpy as jnp
from jax import lax
from jax.experimental import pallas as pl
from jax.experimental.pallas import tpu as pltpu

_TAU = 1.0
_K = 8


def _router_body(x_ref, w_ref, b_ref, g_ref, probs_ref, idx_ref, loss_ref,
                 cnt_ref, *, n_tokens, n_experts):
    pid = pl.program_id(0)

    @pl.when(pid == 0)
    def _init():
        cnt_ref[...] = jnp.zeros_like(cnt_ref)

    z = jnp.dot(x_ref[...], w_ref[...], preferred_element_type=jnp.float32)
    z = (z + b_ref[...] + g_ref[...]) * (1.0 / _TAU)

    iota_f = lax.broadcasted_iota(jnp.int32, z.shape, 1).astype(jnp.float32)
    vals = z
    sel = jnp.zeros(z.shape, jnp.float32)
    top_v = []
    top_i = []
    for _ in range(_K):
        m = jnp.max(vals, axis=1, keepdims=True)
        idx = jnp.min(jnp.where(vals == m, iota_f, float(n_experts)), axis=1,
                      keepdims=True)
        top_v.append(m)
        top_i.append(idx)
        hit = iota_f == idx
        vals = jnp.where(hit, -jnp.inf, vals)
        sel = sel + hit.astype(jnp.float32)

    cnt_ref[...] += jnp.sum(sel, axis=0, keepdims=True)

    vtop = jnp.concatenate(top_v, axis=1)
    itop = jnp.concatenate(top_i, axis=1).astype(jnp.int32)
    e = jnp.exp(vtop - top_v[0])
    probs_ref[...] = e / jnp.sum(e, axis=1, keepdims=True)
    idx_ref[...] = itop

    @pl.when(pid == pl.num_programs(0) - 1)
    def _loss():
        counts = cnt_ref[...]
        total = jnp.float32(n_tokens * _K)
        expert_probs = counts / total
        t = jnp.float32(1.0 / n_experts)
        kl = jnp.sum(t * (jnp.log(t) - jnp.log(expert_probs)),
                     keepdims=True)
        loss_ref[...] = kl.reshape(1, 1) / n_experts


def kernel(x, W, b, gumbel):
    B, S, D = x.shape
    E = W.shape[1]
    T = B * S
    x2 = x.reshape(T, D)
    g2 = gumbel.reshape(T, E)
    b2 = b.reshape(1, E)

    bt = 512
    while T % bt:
        bt //= 2
    grid = T // bt

    body = functools.partial(_router_body, n_tokens=T, n_experts=E)
    probs, idx, loss = pl.pallas_call(
        body,
        grid=(grid,),
        in_specs=[
            pl.BlockSpec((bt, D), lambda i: (i, 0)),
            pl.BlockSpec((D, E), lambda i: (0, 0)),
            pl.BlockSpec((1, E), lambda i: (0, 0)),
            pl.BlockSpec((bt, E), lambda i: (i, 0)),
        ],
        out_specs=[
            pl.BlockSpec((bt, _K), lambda i: (i, 0)),
            pl.BlockSpec((bt, _K), lambda i: (i, 0)),
            pl.BlockSpec((1, 1), lambda i: (0, 0)),
        ],
        out_shape=[
            jax.ShapeDtypeStruct((T, _K), jnp.float32),
            jax.ShapeDtypeStruct((T, _K), jnp.int32),
            jax.ShapeDtypeStruct((1, 1), jnp.float32),
        ],
        scratch_shapes=[pltpu.VMEM((1, E), jnp.float32)],
        compiler_params=pltpu.CompilerParams(
            dimension_semantics=("arbitrary",),
        ),
    )(x2, W, b2, g2)

    return (probs.reshape(B, S, _K), idx.reshape(B, S, _K),
            loss.reshape(()))


# speedup vs baseline: 1.9798x; 1.0969x over previous
import functools

import jax
import jax.numpy as jnp
from jax import lax
from jax.experimental import pallas as pl
from jax.experimental.pallas import tpu as pltpu

_TAU = 1.0
_K = 8


def _router_body(x_ref, w_ref, b_ref, g_ref, probs_ref, idx_ref, loss_ref,
                 cnt_ref, *, n_tokens, n_experts):
    pid = pl.program_id(0)

    @pl.when(pid == 0)
    def _init():
        cnt_ref[...] = jnp.zeros_like(cnt_ref)

    z = jnp.dot(x_ref[...], w_ref[...], preferred_element_type=jnp.float32)
    z = (z + b_ref[...] + g_ref[...]) * (1.0 / _TAU)

    iota_f = lax.broadcasted_iota(jnp.int32, z.shape, 1).astype(jnp.float32)
    vals = z
    sel = jnp.zeros(z.shape, jnp.float32)
    top_v = []
    top_i = []
    for _ in range(_K):
        m = jnp.max(vals, axis=1, keepdims=True)
        idx = jnp.min(jnp.where(vals == m, iota_f, float(n_experts)), axis=1,
                      keepdims=True)
        top_v.append(m)
        top_i.append(idx)
        hit = iota_f == idx
        vals = jnp.where(hit, -jnp.inf, vals)
        sel = sel + hit.astype(jnp.float32)

    cnt_ref[...] += jnp.sum(sel, axis=0, keepdims=True)

    vtop = jnp.concatenate(top_v, axis=1)
    itop = jnp.concatenate(top_i, axis=1).astype(jnp.int32)
    e = jnp.exp(vtop - top_v[0])
    probs_ref[...] = e / jnp.sum(e, axis=1, keepdims=True)
    idx_ref[...] = itop

    @pl.when(pid == pl.num_programs(0) - 1)
    def _loss():
        counts = cnt_ref[...]
        total = jnp.float32(n_tokens * _K)
        expert_probs = counts / total
        t = jnp.float32(1.0 / n_experts)
        kl = jnp.sum(t * (jnp.log(t) - jnp.log(expert_probs)),
                     keepdims=True)
        loss_ref[...] = kl.reshape(1, 1) / n_experts


def kernel(x, W, b, gumbel):
    B, S, D = x.shape
    E = W.shape[1]
    T = B * S
    x2 = x.reshape(T, D)
    g2 = gumbel.reshape(T, E)
    b2 = b.reshape(1, E)

    bt = 1024
    while T % bt:
        bt //= 2
    grid = T // bt

    body = functools.partial(_router_body, n_tokens=T, n_experts=E)
    probs, idx, loss = pl.pallas_call(
        body,
        grid=(grid,),
        in_specs=[
            pl.BlockSpec((bt, D), lambda i: (i, 0)),
            pl.BlockSpec((D, E), lambda i: (0, 0)),
            pl.BlockSpec((1, E), lambda i: (0, 0)),
            pl.BlockSpec((bt, E), lambda i: (i, 0)),
        ],
        out_specs=[
            pl.BlockSpec((bt, _K), lambda i: (i, 0)),
            pl.BlockSpec((bt, _K), lambda i: (i, 0)),
            pl.BlockSpec((1, 1), lambda i: (0, 0)),
        ],
        out_shape=[
            jax.ShapeDtypeStruct((T, _K), jnp.float32),
            jax.ShapeDtypeStruct((T, _K), jnp.int32),
            jax.ShapeDtypeStruct((1, 1), jnp.float32),
        ],
        scratch_shapes=[pltpu.VMEM((1, E), jnp.float32)],
        compiler_params=pltpu.CompilerParams(
            dimension_semantics=("arbitrary",),
        ),
    )(x2, W, b2, g2)

    return (probs.reshape(B, S, _K), idx.reshape(B, S, _K),
            loss.reshape(()))
